# baseline (speedup 1.0000x reference)
import functools

import jax
import jax.numpy as jnp
from jax import lax
from jax.experimental import pallas as pl
from jax.experimental.pallas import tpu as pltpu
from jax.experimental.pallas import tpu_sc as plsc

_VOCAB = 100000
_D = 128
_B = 1024
_H = 200
_N = _B * _H

_NC = 2
_NS = 16
_NW = _NC * _NS
_PER_W = _N // _NW
_CHUNK = 64
_NCHUNK = _PER_W // _CHUNK
_NBUF = 10


def _make_lookup():
    mesh = plsc.VectorSubcoreMesh(core_axis_name="c", subcore_axis_name="s")

    @functools.partial(
        pl.kernel,
        out_type=jax.ShapeDtypeStruct((_N, _D), jnp.float32),
        mesh=mesh,
        scratch_types=[
            pltpu.VMEM((_NCHUNK, _CHUNK), jnp.int32),
            pltpu.VMEM((_NBUF, _CHUNK, _D), jnp.float32),
            pltpu.SemaphoreType.DMA((_NBUF,)),
            pltpu.SemaphoreType.DMA((_NBUF,)),
        ],
    )
    def lookup(idx_hbm, table_hbm, out_hbm, idx_v, rows_v, gsem, ssem):
        wid = lax.axis_index("s") * _NC + lax.axis_index("c")
        base = wid * _PER_W
        pltpu.sync_copy(idx_hbm.at[wid], idx_v)

        def gather_start(chunk, slot):
            pltpu.async_copy(
                table_hbm.at[idx_v.at[chunk]], rows_v.at[slot], gsem.at[slot]
            )

        def gather_wait(chunk, slot):
            pltpu.make_async_copy(
                table_hbm.at[idx_v.at[chunk]], rows_v.at[slot], gsem.at[slot]
            ).wait()

        def scatter_start(chunk, slot):
            pltpu.async_copy(
                rows_v.at[slot],
                out_hbm.at[pl.ds(base + chunk * _CHUNK, _CHUNK)],
                ssem.at[slot],
            )

        def scatter_wait(chunk, slot):
            pltpu.make_async_copy(
                rows_v.at[slot],
                out_hbm.at[pl.ds(base + chunk * _CHUNK, _CHUNK)],
                ssem.at[slot],
            ).wait()

        for b in range(_NBUF - 1):
            gather_start(b, b)

        def outer(g, carry):
            for b in range(_NBUF):
                j = g * _NBUF + b
                prev = (b - 1) % _NBUF

                @pl.when(j + _NBUF - 1 < _NCHUNK)
                def _():
                    @pl.when(j >= 1)
                    def _():
                        scatter_wait(j - 1, prev)

                    gather_start(j + _NBUF - 1, prev)

                gather_wait(j, b)
                scatter_start(j, b)
            return carry

        lax.fori_loop(0, _NCHUNK // _NBUF, outer, 0)

        for b in range(_NBUF):
            last = _NCHUNK - _NBUF + b
            scatter_wait(last, last % _NBUF)

    return lookup


_lookup = _make_lookup()


def kernel(batch, table):
    idx = batch.reshape(_NW, _NCHUNK, _CHUNK).astype(jnp.int32)
    out = _lookup(idx, table)
    return out.reshape(_B, _H, _D)


# speedup vs baseline: 1.0045x; 1.0045x over previous
import functools

import jax
import jax.numpy as jnp
from jax import lax
from jax.experimental import pallas as pl
from jax.experimental.pallas import tpu as pltpu
from jax.experimental.pallas import tpu_sc as plsc

_VOCAB = 100000
_D = 128
_B = 1024
_H = 200
_N = _B * _H

_NC = 2
_NS = 16
_NW = _NC * _NS
_PER_W = _N // _NW
_CHUNK = 64
_NCHUNK = _PER_W // _CHUNK
_NBUF = 10
_LEAD = 5


def _make_lookup():
    mesh = plsc.VectorSubcoreMesh(core_axis_name="c", subcore_axis_name="s")

    @functools.partial(
        pl.kernel,
        out_type=jax.ShapeDtypeStruct((_N, _D), jnp.float32),
        mesh=mesh,
        scratch_types=[
            pltpu.VMEM((_NCHUNK, _CHUNK), jnp.int32),
            pltpu.VMEM((_NBUF, _CHUNK, _D), jnp.float32),
            pltpu.SemaphoreType.DMA((_NBUF,)),
            pltpu.SemaphoreType.DMA((_NBUF,)),
        ],
    )
    def lookup(idx_hbm, table_hbm, out_hbm, idx_v, rows_v, gsem, ssem):
        wid = lax.axis_index("s") * _NC + lax.axis_index("c")
        base = wid * _PER_W
        pltpu.sync_copy(idx_hbm.at[wid], idx_v)

        def gather_start(chunk, slot):
            pltpu.async_copy(
                table_hbm.at[idx_v.at[chunk]], rows_v.at[slot], gsem.at[slot]
            )

        def gather_wait(chunk, slot):
            pltpu.make_async_copy(
                table_hbm.at[idx_v.at[chunk]], rows_v.at[slot], gsem.at[slot]
            ).wait()

        def scatter_start(chunk, slot):
            pltpu.async_copy(
                rows_v.at[slot],
                out_hbm.at[pl.ds(base + chunk * _CHUNK, _CHUNK)],
                ssem.at[slot],
            )

        def scatter_wait(chunk, slot):
            pltpu.make_async_copy(
                rows_v.at[slot],
                out_hbm.at[pl.ds(base + chunk * _CHUNK, _CHUNK)],
                ssem.at[slot],
            ).wait()

        for b in range(_LEAD):
            gather_start(b, b)

        def outer(g, carry):
            for b in range(_NBUF):
                j = g * _NBUF + b
                nxt = j + _LEAD
                slot_n = (b + _LEAD) % _NBUF

                @pl.when(nxt < _NCHUNK)
                def _():
                    @pl.when(nxt - _NBUF >= 0)
                    def _():
                        scatter_wait(nxt - _NBUF, slot_n)

                    gather_start(nxt, slot_n)

                gather_wait(j, b)
                scatter_start(j, b)
            return carry

        lax.fori_loop(0, _NCHUNK // _NBUF, outer, 0)

        for b in range(_NBUF):
            last = _NCHUNK - _NBUF + b
            scatter_wait(last, last % _NBUF)

    return lookup


_lookup = _make_lookup()


def kernel(batch, table):
    idx = batch.reshape(_NW, _NCHUNK, _CHUNK).astype(jnp.int32)
    out = _lookup(idx, table)
    return out.reshape(_B, _H, _D)


# speedup vs baseline: 1.0068x; 1.0022x over previous
import functools

import jax
import jax.numpy as jnp
from jax import lax
from jax.experimental import pallas as pl
from jax.experimental.pallas import tpu as pltpu
from jax.experimental.pallas import tpu_sc as plsc

_VOCAB = 100000
_D = 128
_B = 1024
_H = 200
_N = _B * _H

_NC = 2
_NS = 16
_NW = _NC * _NS
_PER_W = _N // _NW
_CHUNK = 128
_NCHUNK = _PER_W // _CHUNK
_NBUF = 5
_LEAD = 3


def _make_lookup():
    mesh = plsc.VectorSubcoreMesh(core_axis_name="c", subcore_axis_name="s")

    @functools.partial(
        pl.kernel,
        out_type=jax.ShapeDtypeStruct((_N, _D), jnp.float32),
        mesh=mesh,
        scratch_types=[
            pltpu.VMEM((_NCHUNK, _CHUNK), jnp.int32),
            pltpu.VMEM((_NBUF, _CHUNK, _D), jnp.float32),
            pltpu.SemaphoreType.DMA((_NBUF,)),
            pltpu.SemaphoreType.DMA((_NBUF,)),
        ],
    )
    def lookup(idx_hbm, table_hbm, out_hbm, idx_v, rows_v, gsem, ssem):
        wid = lax.axis_index("s") * _NC + lax.axis_index("c")
        base = wid * _PER_W
        pltpu.sync_copy(idx_hbm.at[wid], idx_v)

        def gather_start(chunk, slot):
            pltpu.async_copy(
                table_hbm.at[idx_v.at[chunk]], rows_v.at[slot], gsem.at[slot]
            )

        def gather_wait(chunk, slot):
            pltpu.make_async_copy(
                table_hbm.at[idx_v.at[chunk]], rows_v.at[slot], gsem.at[slot]
            ).wait()

        def scatter_start(chunk, slot):
            pltpu.async_copy(
                rows_v.at[slot],
                out_hbm.at[pl.ds(base + chunk * _CHUNK, _CHUNK)],
                ssem.at[slot],
            )

        def scatter_wait(chunk, slot):
            pltpu.make_async_copy(
                rows_v.at[slot],
                out_hbm.at[pl.ds(base + chunk * _CHUNK, _CHUNK)],
                ssem.at[slot],
            ).wait()

        for b in range(_LEAD):
            gather_start(b, b)

        def outer(g, carry):
            for b in range(_NBUF):
                j = g * _NBUF + b
                nxt = j + _LEAD
                slot_n = (b + _LEAD) % _NBUF

                @pl.when(nxt < _NCHUNK)
                def _():
                    @pl.when(nxt - _NBUF >= 0)
                    def _():
                        scatter_wait(nxt - _NBUF, slot_n)

                    gather_start(nxt, slot_n)

                gather_wait(j, b)
                scatter_start(j, b)
            return carry

        lax.fori_loop(0, _NCHUNK // _NBUF, outer, 0)

        for b in range(_NBUF):
            last = _NCHUNK - _NBUF + b
            scatter_wait(last, last % _NBUF)

    return lookup


_lookup = _make_lookup()


def kernel(batch, table):
    idx = batch.reshape(_NW, _NCHUNK, _CHUNK).astype(jnp.int32)
    out = _lookup(idx, table)
    return out.reshape(_B, _H, _D)


# speedup vs baseline: 1.0132x; 1.0064x over previous
import functools

import jax
import jax.numpy as jnp
from jax import lax
from jax.experimental import pallas as pl
from jax.experimental.pallas import tpu as pltpu
from jax.experimental.pallas import tpu_sc as plsc

_VOCAB = 100000
_D = 128
_B = 1024
_H = 200
_N = _B * _H

_NC = 2
_NS = 16
_NW = _NC * _NS
_PER_W = _N // _NW
_CHUNK = 128
_NCHUNK = _PER_W // _CHUNK
_NBUF = 5
_LEAD = 4


def _make_lookup():
    mesh = plsc.VectorSubcoreMesh(core_axis_name="c", subcore_axis_name="s")

    @functools.partial(
        pl.kernel,
        out_type=jax.ShapeDtypeStruct((_N, _D), jnp.float32),
        mesh=mesh,
        scratch_types=[
            pltpu.VMEM((_NCHUNK, _CHUNK), jnp.int32),
            pltpu.VMEM((_NBUF, _CHUNK, _D), jnp.float32),
            pltpu.SemaphoreType.DMA((_NBUF,)),
            pltpu.SemaphoreType.DMA((_NBUF,)),
        ],
    )
    def lookup(idx_hbm, table_hbm, out_hbm, idx_v, rows_v, gsem, ssem):
        wid = lax.axis_index("s") * _NC + lax.axis_index("c")
        base = wid * _PER_W
        pltpu.sync_copy(idx_hbm.at[wid], idx_v)

        def gather_start(chunk, slot):
            pltpu.async_copy(
                table_hbm.at[idx_v.at[chunk]], rows_v.at[slot], gsem.at[slot]
            )

        def gather_wait(chunk, slot):
            pltpu.make_async_copy(
                table_hbm.at[idx_v.at[chunk]], rows_v.at[slot], gsem.at[slot]
            ).wait()

        def scatter_start(chunk, slot):
            pltpu.async_copy(
                rows_v.at[slot],
                out_hbm.at[pl.ds(base + chunk * _CHUNK, _CHUNK)],
                ssem.at[slot],
            )

        def scatter_wait(chunk, slot):
            pltpu.make_async_copy(
                rows_v.at[slot],
                out_hbm.at[pl.ds(base + chunk * _CHUNK, _CHUNK)],
                ssem.at[slot],
            ).wait()

        for b in range(_LEAD):
            gather_start(b, b)

        def outer(g, carry):
            for b in range(_NBUF):
                j = g * _NBUF + b
                nxt = j + _LEAD
                slot_n = (b + _LEAD) % _NBUF

                @pl.when(nxt < _NCHUNK)
                def _():
                    @pl.when(nxt - _NBUF >= 0)
                    def _():
                        scatter_wait(nxt - _NBUF, slot_n)

                    gather_start(nxt, slot_n)

                gather_wait(j, b)
                scatter_start(j, b)
            return carry

        lax.fori_loop(0, _NCHUNK // _NBUF, outer, 0)

        for b in range(_NBUF):
            last = _NCHUNK - _NBUF + b
            scatter_wait(last, last % _NBUF)

    return lookup


_lookup = _make_lookup()


def kernel(batch, table):
    idx = batch.reshape(_NW, _NCHUNK, _CHUNK).astype(jnp.int32)
    out = _lookup(idx, table)
    return out.reshape(_B, _H, _D)
